# baseline (speedup 1.0000x reference)
import functools

import jax
import jax.numpy as jnp
from jax import lax
from jax.experimental import pallas as pl
from jax.experimental.pallas import tpu as pltpu
from jax.experimental.pallas import tpu_sc as plsc

VOLUME_DIM = 128
F = 8
HIDDEN = 32
M = VOLUME_DIM ** 3
N = 786432

NSC = 2
NTILE = 16
NW = NSC * NTILE

HALF = M // NSC
NPASS = 6
CHUNK_STEP = 174848
CHUNKS = [CHUNK_STEP] * (NPASS - 1) + [HALF - (NPASS - 1) * CHUNK_STEP]
CHUNK_ALLOC = CHUNK_STEP + NTILE

PPT_SC = N // NTILE
WIN = 4096
NWIN = PPT_SC // WIN
SEG = 128
NROWS = (WIN + SEG + 127) // 128
NBUF = 8

PPT_G = N // NW
NSEG_G = PPT_G // SEG


def _scatter_body(mem_hbm, counts_hbm, values_hbm, idx_hbm,
                  fv_hbm, cv_hbm,
                  idxbuf, lidx_buf, posq_buf, rows_buf, ones_buf,
                  feat_sh, cnt_sh, gsem):
    c = lax.axis_index("c")
    s = lax.axis_index("s")
    lanes = lax.iota(jnp.int32, 16)
    wid = s * NSC + c

    for t in range(SEG // 16):
        ones_buf[pl.ds(t * 16, 16)] = jnp.ones((16,), jnp.float32)

    for p in range(NPASS):
        chunk = CHUNKS[p]
        piece = chunk // NTILE
        lo = c * HALF + p * CHUNK_STEP
        hi = lo + chunk

        pltpu.sync_copy(mem_hbm.at[pl.ds(lo + s * piece, piece)],
                        feat_sh.at[pl.ds(s * piece, piece)])
        pltpu.sync_copy(counts_hbm.at[pl.ds(lo + s * piece, piece)],
                        cnt_sh.at[pl.ds(s * piece, piece)])
        plsc.subcore_barrier()

        def win_body(w, _, s=s, lo=lo, hi=hi, chunk=chunk):
            base = s * PPT_SC + w * WIN
            pltpu.sync_copy(idx_hbm.at[pl.ds(base, WIN)], idxbuf)

            def filt(i, ptr_vec):
                v = idxbuf[pl.ds(i * 16, 16)]
                m = (v >= lo) & (v < hi)
                pf = plsc.cumsum(m.astype(jnp.int32))
                slot = ptr_vec + pf - 1
                row = lax.shift_right_logical(slot, 7)
                col = lax.bitwise_and(slot, SEG - 1)
                plsc.store_scatter(lidx_buf, [row, col], v - lo, mask=m)
                plsc.store_scatter(posq_buf, [row, col],
                                   base + i * 16 + lanes, mask=m)
                return ptr_vec + plsc.all_reduce_population_count(m)

            ptr_vec = lax.fori_loop(0, WIN // 16, filt,
                                    jnp.zeros((16,), jnp.int32))
            k = jnp.max(ptr_vec)
            nseg = (k + SEG - 1) // SEG
            kpad = nseg * SEG

            def padb(t, _):
                slot = k + t * 16 + lanes
                pm = slot < kpad
                row = lax.shift_right_logical(slot, 7)
                col = lax.bitwise_and(slot, SEG - 1)
                plsc.store_scatter(lidx_buf, [row, col], chunk + lanes,
                                   mask=pm)
                plsc.store_scatter(posq_buf, [row, col],
                                   wid * SEG + t * 16 + lanes, mask=pm)
                return 0
            lax.fori_loop(0, (kpad - k + 15) // 16, padb, 0)

            def fire(j):
                jm = lax.rem(j, NBUF)
                pltpu.async_copy(values_hbm.at[posq_buf.at[j]],
                                 rows_buf.at[jm], gsem)

            def prol(j, _):
                fire(j)
                return 0
            lax.fori_loop(0, jnp.minimum(nseg, NBUF), prol, 0)

            def segb(j, _):
                jm = lax.rem(j, NBUF)
                pltpu.make_async_copy(values_hbm.at[posq_buf.at[j]],
                                      rows_buf.at[jm], gsem).wait()
                pltpu.sync_copy(rows_buf.at[jm], feat_sh.at[lidx_buf.at[j]],
                                add=True)
                pltpu.sync_copy(ones_buf, cnt_sh.at[lidx_buf.at[j]], add=True)

                @pl.when(j + NBUF < nseg)
                def _():
                    fire(j + NBUF)
                return 0
            lax.fori_loop(0, nseg, segb, 0)
            return 0

        lax.fori_loop(0, NWIN, win_body, 0)
        plsc.subcore_barrier()

        pltpu.sync_copy(feat_sh.at[pl.ds(s * piece, piece)],
                        fv_hbm.at[pl.ds(lo + s * piece, piece)])
        pltpu.sync_copy(cnt_sh.at[pl.ds(s * piece, piece)],
                        cv_hbm.at[pl.ds(lo + s * piece, piece)])
        plsc.subcore_barrier()


def _gather_body(fv_hbm, cv_hbm, idx2d_hbm, gfn_hbm,
                 idxb, growsb, gcntb, invb, outb, s1, s2):
    c = lax.axis_index("c")
    s = lax.axis_index("s")
    wid = s * NSC + c
    lanes = lax.iota(jnp.int32, 16)
    row16 = lax.shift_right_logical(lanes, 3)
    col16 = lax.bitwise_and(lanes, 7)

    pltpu.sync_copy(idx2d_hbm.at[pl.ds(wid * NSEG_G, NSEG_G)], idxb)
    base_row = wid * PPT_G * F // SEG

    def fire(j):
        jm = lax.rem(j, NBUF)
        pltpu.async_copy(fv_hbm.at[idxb.at[j]], growsb.at[jm], s1)
        pltpu.async_copy(cv_hbm.at[idxb.at[j]], gcntb.at[jm], s2)

    for j in range(NBUF):
        fire(j)

    def segb(j, _):
        jm = lax.rem(j, NBUF)
        pltpu.make_async_copy(fv_hbm.at[idxb.at[j]], growsb.at[jm], s1).wait()
        pltpu.make_async_copy(cv_hbm.at[idxb.at[j]], gcntb.at[jm], s2).wait()
        for q in range(SEG // 16):
            c16 = gcntb[jm, pl.ds(q * 16, 16)]
            invb[pl.ds(q * 16, 16)] = 1.0 / jnp.maximum(c16, 1.0)
        for r in range(SEG * F // 16 // 8):
            for t in range(8):
                fr = r * 8 + t
                v = plsc.load_gather(growsb.at[jm],
                                     [2 * fr + row16, col16])
                iv = plsc.load_gather(invb, [2 * fr + row16])
                outb[r, pl.ds(t * 16, 16)] = v * iv
        pltpu.sync_copy(outb, gfn_hbm.at[pl.ds(base_row + j * F, F)])

        @pl.when(j + NBUF < NSEG_G)
        def _():
            fire(j + NBUF)
        return 0
    lax.fori_loop(0, NSEG_G, segb, 0)


_sc_mesh = dict(core_axis_name="c", subcore_axis_name="s")
_sc_params = pltpu.CompilerParams(needs_layout_passes=False,
                                  use_tc_tiling_on_sc=False)

_scatter_call = pl.kernel(
    _scatter_body,
    out_type=[jax.ShapeDtypeStruct((M, F), jnp.float32),
              jax.ShapeDtypeStruct((M,), jnp.float32)],
    mesh=plsc.VectorSubcoreMesh(**_sc_mesh),
    compiler_params=_sc_params,
    scratch_types=[
        pltpu.VMEM((WIN,), jnp.int32),
        pltpu.VMEM((NROWS, SEG), jnp.int32),
        pltpu.VMEM((NROWS, SEG), jnp.int32),
        pltpu.VMEM((NBUF, SEG, F), jnp.float32),
        pltpu.VMEM((SEG,), jnp.float32),
        pltpu.VMEM_SHARED((CHUNK_ALLOC, F), jnp.float32),
        pltpu.VMEM_SHARED((CHUNK_ALLOC,), jnp.float32),
        pltpu.SemaphoreType.DMA,
    ],
)

_gather_call = pl.kernel(
    _gather_body,
    out_type=[jax.ShapeDtypeStruct((N * F // SEG, SEG), jnp.float32)],
    mesh=plsc.VectorSubcoreMesh(**_sc_mesh),
    compiler_params=_sc_params,
    scratch_types=[
        pltpu.VMEM((NSEG_G, SEG), jnp.int32),
        pltpu.VMEM((NBUF, SEG, F), jnp.float32),
        pltpu.VMEM((NBUF, SEG), jnp.float32),
        pltpu.VMEM((SEG,), jnp.float32),
        pltpu.VMEM((F, SEG), jnp.float32),
        pltpu.SemaphoreType.DMA,
        pltpu.SemaphoreType.DMA,
    ],
)

RB = 2048
NROW_X = N * F // SEG
NROW_Z = N // 16


def _mlp_body(x_ref, w1_ref, b1_ref, wt_ref, wo_ref, bt_ref, bo_ref,
              t_ref, o_ref):
    h = jnp.tanh(jnp.dot(x_ref[...], w1_ref[...],
                         preferred_element_type=jnp.float32) + b1_ref[...])
    t_ref[...] = jnp.tanh(jnp.dot(h, wt_ref[...],
                                  preferred_element_type=jnp.float32)
                          + bt_ref[...])
    o_ref[...] = jax.nn.sigmoid(jnp.dot(h, wo_ref[...],
                                        preferred_element_type=jnp.float32)
                                + bo_ref[...])


_mlp_call = pl.pallas_call(
    _mlp_body,
    grid=(NROW_X // RB,),
    in_specs=[
        pl.BlockSpec((RB, SEG), lambda i: (i, 0)),
        pl.BlockSpec((SEG, 16 * HIDDEN), lambda i: (0, 0)),
        pl.BlockSpec((1, 16 * HIDDEN), lambda i: (0, 0)),
        pl.BlockSpec((16 * HIDDEN, 16), lambda i: (0, 0)),
        pl.BlockSpec((16 * HIDDEN, 16), lambda i: (0, 0)),
        pl.BlockSpec((1, 16), lambda i: (0, 0)),
        pl.BlockSpec((1, 16), lambda i: (0, 0)),
    ],
    out_specs=[
        pl.BlockSpec((RB, 16), lambda i: (i, 0)),
        pl.BlockSpec((RB, 16), lambda i: (i, 0)),
    ],
    out_shape=[jax.ShapeDtypeStruct((NROW_Z, 16), jnp.float32),
               jax.ShapeDtypeStruct((NROW_Z, 16), jnp.float32)],
)


def kernel(mem, counts, values, indices, W1, b1, W2t, b2t, W2o, b2o):
    fv, cv = _scatter_call(mem, counts, values, indices)
    gfn, = _gather_call(fv, cv, indices.reshape(N // SEG, SEG))
    eye = jnp.eye(16, dtype=jnp.float32)
    w1k = jnp.kron(eye, W1)
    b1k = jnp.tile(b1, 16)[None, :]
    wtk = jnp.kron(eye, W2t)
    wok = jnp.kron(eye, W2o)
    btk = jnp.tile(b2t, 16)[None, :]
    bok = jnp.tile(b2o, 16)[None, :]
    t2d, o2d = _mlp_call(gfn, w1k, b1k, wtk, wok, btk, bok)
    return fv, cv, t2d.reshape(N, 1), o2d.reshape(N, 1)


# speedup vs baseline: 1.9934x; 1.9934x over previous
import functools

import jax
import jax.numpy as jnp
from jax import lax
from jax.experimental import pallas as pl
from jax.experimental.pallas import tpu as pltpu
from jax.experimental.pallas import tpu_sc as plsc

VOLUME_DIM = 128
F = 8
HIDDEN = 32
M = VOLUME_DIM ** 3
N = 786432

NSC = 2
NTILE = 16
NW = NSC * NTILE

HALF = M // NSC
NPASS = 6
CHUNK_STEP = 174848
CHUNKS = [CHUNK_STEP] * (NPASS - 1) + [HALF - (NPASS - 1) * CHUNK_STEP]
CHUNK_ALLOC = CHUNK_STEP + NTILE

PPT_SC = N // NTILE
WIN = 4096
NWIN = PPT_SC // WIN
SEG = 128
NROWS = (WIN + SEG + 127) // 128
NBUF = 8
ZROWS = 512
BB = 16

PPT_G = N // NW
NSEG_G = PPT_G // SEG


def _scatter_body(counts_hbm, values_hbm, idx_hbm,
                  dv_hbm, cv_hbm,
                  idxbuf, lidx_buf, posq_buf, rows_buf, ones_buf, zbuf,
                  feat_sh, cnt_sh, gsem, isem):
    c = lax.axis_index("c")
    s = lax.axis_index("s")
    lanes = lax.iota(jnp.int32, 16)
    wid = s * NSC + c

    for t in range(SEG // 16):
        ones_buf[pl.ds(t * 16, 16)] = jnp.ones((16,), jnp.float32)

    def zfill(r, _):
        fp = r * 16 + lanes
        plsc.store_scatter(zbuf, [lax.shift_right_logical(fp, 3),
                                  lax.bitwise_and(fp, 7)],
                           jnp.zeros((16,), jnp.float32))
        return 0
    lax.fori_loop(0, ZROWS * F // 16, zfill, 0)

    for p in range(NPASS):
        chunk = CHUNKS[p]
        piece = chunk // NTILE
        lo = c * HALF + p * CHUNK_STEP
        hi = lo + chunk

        nz = piece // ZROWS
        for q in range(nz):
            pltpu.sync_copy(zbuf,
                            feat_sh.at[pl.ds(s * piece + q * ZROWS, ZROWS)])
        rem = piece - nz * ZROWS
        if rem:
            pltpu.sync_copy(zbuf.at[pl.ds(0, rem)],
                            feat_sh.at[pl.ds(s * piece + nz * ZROWS, rem)])
        pltpu.sync_copy(counts_hbm.at[pl.ds(lo + s * piece, piece)],
                        cnt_sh.at[pl.ds(s * piece, piece)])
        plsc.subcore_barrier()

        def fire_win(w, s=s):
            base = s * PPT_SC + w * WIN
            pltpu.async_copy(idx_hbm.at[pl.ds(base, WIN)],
                             idxbuf.at[lax.rem(w, 2)], isem)

        fire_win(jnp.int32(0))

        def win_body(w, _, s=s, lo=lo, hi=hi, chunk=chunk):
            base = s * PPT_SC + w * WIN
            wm = lax.rem(w, 2)
            pltpu.make_async_copy(idx_hbm.at[pl.ds(base, WIN)],
                                  idxbuf.at[wm], isem).wait()

            @pl.when(w + 1 < NWIN)
            def _():
                fire_win(w + 1)

            def filt(i, ptr_vec):
                v = idxbuf[wm, pl.ds(i * 16, 16)]
                m = (v >= lo) & (v < hi)
                pf = plsc.cumsum(m.astype(jnp.int32))
                slot = ptr_vec + pf - 1
                row = lax.shift_right_logical(slot, 7)
                col = lax.bitwise_and(slot, SEG - 1)
                plsc.store_scatter(lidx_buf, [row, col], v - lo, mask=m)
                plsc.store_scatter(posq_buf, [row, col],
                                   base + i * 16 + lanes, mask=m)
                return ptr_vec + plsc.all_reduce_population_count(m)

            ptr_vec = lax.fori_loop(0, WIN // 16, filt,
                                    jnp.zeros((16,), jnp.int32))
            k = jnp.max(ptr_vec)
            nseg = (k + SEG - 1) // SEG
            kpad = nseg * SEG

            def padb(t, _):
                slot = k + t * 16 + lanes
                pm = slot < kpad
                row = lax.shift_right_logical(slot, 7)
                col = lax.bitwise_and(slot, SEG - 1)
                plsc.store_scatter(lidx_buf, [row, col], chunk + lanes,
                                   mask=pm)
                plsc.store_scatter(posq_buf, [row, col],
                                   wid * SEG + t * 16 + lanes, mask=pm)
                return 0
            lax.fori_loop(0, (kpad - k + 15) // 16, padb, 0)

            def fire(j):
                jm = lax.rem(j, NBUF)
                pltpu.async_copy(values_hbm.at[posq_buf.at[j]],
                                 rows_buf.at[jm], gsem)

            def prol(j, _):
                fire(j)
                return 0
            lax.fori_loop(0, jnp.minimum(nseg, NBUF), prol, 0)

            def segb(j, _):
                jm = lax.rem(j, NBUF)
                pltpu.make_async_copy(values_hbm.at[posq_buf.at[j]],
                                      rows_buf.at[jm], gsem).wait()
                pltpu.sync_copy(rows_buf.at[jm], feat_sh.at[lidx_buf.at[j]],
                                add=True)
                pltpu.sync_copy(ones_buf, cnt_sh.at[lidx_buf.at[j]], add=True)

                @pl.when(j + NBUF < nseg)
                def _():
                    fire(j + NBUF)
                return 0
            lax.fori_loop(0, nseg, segb, 0)
            return 0

        lax.fori_loop(0, NWIN, win_body, 0)
        plsc.subcore_barrier()

        pltpu.sync_copy(feat_sh.at[pl.ds(s * piece, piece)],
                        dv_hbm.at[pl.ds(lo + s * piece, piece)])
        pltpu.sync_copy(cnt_sh.at[pl.ds(s * piece, piece)],
                        cv_hbm.at[pl.ds(lo + s * piece, piece)])
        plsc.subcore_barrier()


def _fmt_body(dv_hbm, mem3_hbm, fv3_hbm, fvr_hbm,
              dvb, memb, sumb, rowsb, si):
    c = lax.axis_index("c")
    s = lax.axis_index("s")
    wid = s * NSC + c
    lanes = lax.iota(jnp.int32, 16)
    zl = lanes * 0
    bpt = M // SEG // NW
    nbat = bpt // BB
    blk0 = wid * bpt

    def fire(t):
        tm = lax.rem(t, 2)
        pltpu.async_copy(dv_hbm.at[pl.ds((blk0 + t * BB) * SEG, BB * SEG)],
                         dvb.at[tm], si)
        pltpu.async_copy(mem3_hbm.at[pl.ds(blk0 + t * BB, BB)],
                         memb.at[tm], si)

    fire(0)

    def batch(t, _):
        tm = lax.rem(t, 2)
        pltpu.make_async_copy(dv_hbm.at[pl.ds((blk0 + t * BB) * SEG, BB * SEG)],
                              dvb.at[tm], si).wait()
        pltpu.make_async_copy(mem3_hbm.at[pl.ds(blk0 + t * BB, BB)],
                              memb.at[tm], si).wait()

        @pl.when(t + 1 < nbat)
        def _():
            fire(t + 1)

        def blksum(i, _):
            b = lax.shift_right_logical(i, 6)
            g = lax.bitwise_and(i, 63)
            f = lax.shift_right_logical(g, 3)
            cc = lax.bitwise_and(g, 7) * 16
            rowv = b * SEG + cc + lanes
            v = plsc.load_gather(dvb.at[tm], [rowv, zl + f])
            mv = memb[tm, b, f, pl.ds(cc, 16)]
            sv = v + mv
            sumb[b, f, pl.ds(cc, 16)] = sv
            plsc.store_scatter(rowsb, [rowv, zl + f], sv)
            return 0
        lax.fori_loop(0, BB * F * SEG // 16, blksum, 0)

        pltpu.sync_copy(sumb, fv3_hbm.at[pl.ds(blk0 + t * BB, BB)])
        pltpu.sync_copy(rowsb,
                        fvr_hbm.at[pl.ds((blk0 + t * BB) * SEG, BB * SEG)])
        return 0
    lax.fori_loop(0, nbat, batch, 0)


def _gather_body(fv_hbm, cv_hbm, idx2d_hbm, gfn_hbm,
                 idxb, growsb, gcntb, invb, outb, s1, s2):
    c = lax.axis_index("c")
    s = lax.axis_index("s")
    wid = s * NSC + c
    lanes = lax.iota(jnp.int32, 16)
    row16 = lax.shift_right_logical(lanes, 3)
    col16 = lax.bitwise_and(lanes, 7)

    pltpu.sync_copy(idx2d_hbm.at[pl.ds(wid * NSEG_G, NSEG_G)], idxb)
    base_row = wid * PPT_G * F // SEG

    def fire(j):
        jm = lax.rem(j, NBUF)
        pltpu.async_copy(fv_hbm.at[idxb.at[j]], growsb.at[jm], s1)
        pltpu.async_copy(cv_hbm.at[idxb.at[j]], gcntb.at[jm], s2)

    for j in range(NBUF):
        fire(j)

    def segb(j, _):
        jm = lax.rem(j, NBUF)
        pltpu.make_async_copy(fv_hbm.at[idxb.at[j]], growsb.at[jm], s1).wait()
        pltpu.make_async_copy(cv_hbm.at[idxb.at[j]], gcntb.at[jm], s2).wait()
        for q in range(SEG // 16):
            c16 = gcntb[jm, pl.ds(q * 16, 16)]
            invb[pl.ds(q * 16, 16)] = 1.0 / jnp.maximum(c16, 1.0)
        for r in range(SEG * F // 16 // 8):
            for t in range(8):
                fr = r * 8 + t
                v = plsc.load_gather(growsb.at[jm],
                                     [2 * fr + row16, col16])
                iv = plsc.load_gather(invb, [2 * fr + row16])
                outb[r, pl.ds(t * 16, 16)] = v * iv
        pltpu.sync_copy(outb, gfn_hbm.at[pl.ds(base_row + j * F, F)])

        @pl.when(j + NBUF < NSEG_G)
        def _():
            fire(j + NBUF)
        return 0
    lax.fori_loop(0, NSEG_G, segb, 0)


_sc_mesh = dict(core_axis_name="c", subcore_axis_name="s")
_sc_params = pltpu.CompilerParams(needs_layout_passes=False,
                                  use_tc_tiling_on_sc=False)

_scatter_call = pl.kernel(
    _scatter_body,
    out_type=[jax.ShapeDtypeStruct((M, F), jnp.float32),
              jax.ShapeDtypeStruct((M,), jnp.float32)],
    mesh=plsc.VectorSubcoreMesh(**_sc_mesh),
    compiler_params=_sc_params,
    scratch_types=[
        pltpu.VMEM((2, WIN), jnp.int32),
        pltpu.VMEM((NROWS, SEG), jnp.int32),
        pltpu.VMEM((NROWS, SEG), jnp.int32),
        pltpu.VMEM((NBUF, SEG, F), jnp.float32),
        pltpu.VMEM((SEG,), jnp.float32),
        pltpu.VMEM((ZROWS, F), jnp.float32),
        pltpu.VMEM_SHARED((CHUNK_ALLOC, F), jnp.float32),
        pltpu.VMEM_SHARED((CHUNK_ALLOC,), jnp.float32),
        pltpu.SemaphoreType.DMA,
        pltpu.SemaphoreType.DMA,
    ],
)

_fmt_call = pl.kernel(
    _fmt_body,
    out_type=[jax.ShapeDtypeStruct((M // SEG, F, SEG), jnp.float32),
              jax.ShapeDtypeStruct((M, F), jnp.float32)],
    mesh=plsc.VectorSubcoreMesh(**_sc_mesh),
    compiler_params=_sc_params,
    scratch_types=[
        pltpu.VMEM((2, BB * SEG, F), jnp.float32),
        pltpu.VMEM((2, BB, F, SEG), jnp.float32),
        pltpu.VMEM((BB, F, SEG), jnp.float32),
        pltpu.VMEM((BB * SEG, F), jnp.float32),
        pltpu.SemaphoreType.DMA,
    ],
)

_gather_call = pl.kernel(
    _gather_body,
    out_type=[jax.ShapeDtypeStruct((N * F // SEG, SEG), jnp.float32)],
    mesh=plsc.VectorSubcoreMesh(**_sc_mesh),
    compiler_params=_sc_params,
    scratch_types=[
        pltpu.VMEM((NSEG_G, SEG), jnp.int32),
        pltpu.VMEM((NBUF, SEG, F), jnp.float32),
        pltpu.VMEM((NBUF, SEG), jnp.float32),
        pltpu.VMEM((SEG,), jnp.float32),
        pltpu.VMEM((F, SEG), jnp.float32),
        pltpu.SemaphoreType.DMA,
        pltpu.SemaphoreType.DMA,
    ],
)

RB = 2048
NROW_X = N * F // SEG
NROW_Z = N // 16


def _mlp_body(x_ref, w1_ref, b1_ref, wt_ref, wo_ref, bt_ref, bo_ref,
              t_ref, o_ref):
    h = jnp.tanh(jnp.dot(x_ref[...], w1_ref[...],
                         preferred_element_type=jnp.float32) + b1_ref[...])
    t_ref[...] = jnp.tanh(jnp.dot(h, wt_ref[...],
                                  preferred_element_type=jnp.float32)
                          + bt_ref[...])
    o_ref[...] = jax.nn.sigmoid(jnp.dot(h, wo_ref[...],
                                        preferred_element_type=jnp.float32)
                                + bo_ref[...])


_mlp_call = pl.pallas_call(
    _mlp_body,
    grid=(NROW_X // RB,),
    in_specs=[
        pl.BlockSpec((RB, SEG), lambda i: (i, 0)),
        pl.BlockSpec((SEG, 16 * HIDDEN), lambda i: (0, 0)),
        pl.BlockSpec((1, 16 * HIDDEN), lambda i: (0, 0)),
        pl.BlockSpec((16 * HIDDEN, 16), lambda i: (0, 0)),
        pl.BlockSpec((16 * HIDDEN, 16), lambda i: (0, 0)),
        pl.BlockSpec((1, 16), lambda i: (0, 0)),
        pl.BlockSpec((1, 16), lambda i: (0, 0)),
    ],
    out_specs=[
        pl.BlockSpec((RB, 16), lambda i: (i, 0)),
        pl.BlockSpec((RB, 16), lambda i: (i, 0)),
    ],
    out_shape=[jax.ShapeDtypeStruct((NROW_Z, 16), jnp.float32),
               jax.ShapeDtypeStruct((NROW_Z, 16), jnp.float32)],
)


def kernel(mem, counts, values, indices, W1, b1, W2t, b2t, W2o, b2o):
    dv, cv = _scatter_call(counts, values, indices)
    mem3 = mem.reshape(M // SEG, SEG, F).transpose(0, 2, 1)
    fv3, fvr = _fmt_call(dv, mem3)
    fv = fv3.transpose(0, 2, 1).reshape(M, F)
    gfn, = _gather_call(fvr, cv, indices.reshape(N // SEG, SEG))
    eye = jnp.eye(16, dtype=jnp.float32)
    w1k = jnp.kron(eye, W1)
    b1k = jnp.tile(b1, 16)[None, :]
    wtk = jnp.kron(eye, W2t)
    wok = jnp.kron(eye, W2o)
    btk = jnp.tile(b2t, 16)[None, :]
    bok = jnp.tile(b2o, 16)[None, :]
    t2d, o2d = _mlp_call(gfn, w1k, b1k, wtk, wok, btk, bok)
    return fv, cv, t2d.reshape(N, 1), o2d.reshape(N, 1)
